# Initial kernel scaffold; baseline (speedup 1.0000x reference)
import functools

import jax
import jax.numpy as jnp
from jax import lax
from jax.experimental import pallas as pl
from jax.experimental.pallas import tpu as pltpu
from jax.experimental.pallas import tpu_sc as plsc

PIXEL_SCALE = 0.2619
N_VERTS = 32768
N = 2 * N_VERTS
NROWS = 2 * N
TOTAL_NNZ = 2 * N * 16

NC = 2
NS = 16
L = 16
NW = NC * NS
CHUNK = TOTAL_NNZ // NW
BLK = 8192
NBLK = CHUNK // BLK
ACC_SLICE = NROWS // NS


def _sc_partial(x_flat, vals, rows, cols):
    mesh = plsc.VectorSubcoreMesh(core_axis_name="c", subcore_axis_name="s")

    @functools.partial(
        pl.kernel,
        out_type=jax.ShapeDtypeStruct((NC, NROWS), jnp.float32),
        mesh=mesh,
        scratch_types=[
            pltpu.VMEM((N,), jnp.float32),
            pltpu.VMEM((BLK,), jnp.int32),
            pltpu.VMEM((BLK,), jnp.float32),
            pltpu.VMEM((BLK,), jnp.int32),
            pltpu.VMEM((BLK,), jnp.float32),
            pltpu.VMEM_SHARED((NROWS,), jnp.float32),
        ],
    )
    def k(x_hbm, vals_hbm, rows_hbm, cols_hbm, out_hbm,
          x_v, cols_v, vals_v, rows_v, prod_v, acc_sh):
        c = lax.axis_index("c")
        s = lax.axis_index("s")
        wid = c * NS + s

        pltpu.sync_copy(x_hbm, x_v)

        def zero_body(i, carry):
            prod_v[pl.ds(i * L, L)] = jnp.zeros((L,), jnp.float32)
            return carry
        lax.fori_loop(0, BLK // L, zero_body, 0)
        pltpu.sync_copy(prod_v, acc_sh.at[pl.ds(s * ACC_SLICE, ACC_SLICE)])
        plsc.subcore_barrier()

        base = wid * CHUNK

        def blk_body(b, carry):
            off = base + b * BLK
            pltpu.sync_copy(cols_hbm.at[pl.ds(off, BLK)], cols_v)
            pltpu.sync_copy(vals_hbm.at[pl.ds(off, BLK)], vals_v)
            pltpu.sync_copy(rows_hbm.at[pl.ds(off, BLK)], rows_v)

            def prod_body(i, inner):
                sl = pl.ds(i * L, L)
                xg = plsc.load_gather(x_v, [cols_v[sl]])
                prod_v[sl] = vals_v[sl] * xg
                return inner
            lax.fori_loop(0, BLK // L, prod_body, 0)

            pltpu.sync_copy(prod_v, acc_sh.at[rows_v], add=True)
            return carry
        lax.fori_loop(0, NBLK, blk_body, 0)

        plsc.subcore_barrier()
        pltpu.sync_copy(acc_sh.at[pl.ds(s * ACC_SLICE, ACC_SLICE)], prod_v)
        pltpu.sync_copy(prod_v, out_hbm.at[c, pl.ds(s * ACC_SLICE, ACC_SLICE)])

    return k(x_flat, vals, rows, cols)


def _combine(partial):
    p = partial.reshape(NC, NROWS // 128, 128)

    def body(p_ref, o_ref):
        o_ref[...] = (p_ref[0] + p_ref[1]) * (1.0 / PIXEL_SCALE)

    out = pl.pallas_call(
        body,
        out_shape=jax.ShapeDtypeStruct((NROWS // 128, 128), jnp.float32),
    )(p)
    return out.reshape(2, 2, N_VERTS)


def kernel(x, vals, rows, cols):
    x_flat = x.reshape(-1)
    partial = _sc_partial(x_flat, vals, rows, cols)
    grad = _combine(partial)
    return jnp.moveaxis(grad, 0, -1)


# speedup vs baseline: 214.1730x; 214.1730x over previous
import functools

import jax
import jax.numpy as jnp
from jax import lax
from jax.experimental import pallas as pl
from jax.experimental.pallas import tpu as pltpu
from jax.experimental.pallas import tpu_sc as plsc

PIXEL_SCALE = 0.2619
N_VERTS = 32768
N = 2 * N_VERTS
NROWS = 2 * N
TOTAL_NNZ = 2 * N * 16

NC = 2
NS = 16
L = 16
NW = NC * NS
CHUNK = TOTAL_NNZ // NW
BLK = 8192
NBLK = CHUNK // BLK
ACC_SLICE = NROWS // NS


def _sc_partial(x_flat, vals, rows, cols):
    mesh = plsc.VectorSubcoreMesh(core_axis_name="c", subcore_axis_name="s")

    @functools.partial(
        pl.kernel,
        out_type=jax.ShapeDtypeStruct((NC, NROWS), jnp.float32),
        mesh=mesh,
        scratch_types=[
            pltpu.VMEM((N,), jnp.float32),
            pltpu.VMEM((BLK,), jnp.int32),
            pltpu.VMEM((BLK,), jnp.float32),
            pltpu.VMEM((BLK,), jnp.int32),
            pltpu.VMEM((BLK,), jnp.float32),
            pltpu.VMEM_SHARED((NROWS,), jnp.float32),
        ],
        compiler_params=pltpu.CompilerParams(needs_layout_passes=False),
    )
    def k(x_hbm, vals_hbm, rows_hbm, cols_hbm, out_hbm,
          x_v, cols_v, vals_v, rows_v, prod_v, acc_sh):
        c = lax.axis_index("c")
        s = lax.axis_index("s")
        wid = c * NS + s

        pltpu.sync_copy(x_hbm, x_v)

        def zero_body(i, carry):
            prod_v[pl.ds(i * L, L)] = jnp.zeros((L,), jnp.float32)
            return carry
        lax.fori_loop(0, BLK // L, zero_body, 0)
        pltpu.sync_copy(prod_v, acc_sh.at[pl.ds(s * ACC_SLICE, ACC_SLICE)])
        plsc.subcore_barrier()

        base = wid * CHUNK

        def blk_body(b, carry):
            off = base + b * BLK
            pltpu.sync_copy(cols_hbm.at[pl.ds(off, BLK)], cols_v)
            pltpu.sync_copy(vals_hbm.at[pl.ds(off, BLK)], vals_v)
            pltpu.sync_copy(rows_hbm.at[pl.ds(off, BLK)], rows_v)

            def prod_body(i, inner):
                sl = pl.ds(i * L, L)
                xg = plsc.load_gather(x_v, [cols_v[sl]])
                prod_v[sl] = vals_v[sl] * xg
                return inner
            lax.fori_loop(0, BLK // L, prod_body, 0)

            pltpu.sync_copy(prod_v, acc_sh.at[rows_v], add=True)
            return carry
        lax.fori_loop(0, NBLK, blk_body, 0)

        plsc.subcore_barrier()
        pltpu.sync_copy(acc_sh.at[pl.ds(s * ACC_SLICE, ACC_SLICE)], prod_v)
        pltpu.sync_copy(prod_v, out_hbm.at[c, pl.ds(s * ACC_SLICE, ACC_SLICE)])

    return k(x_flat, vals, rows, cols)


def _combine(partial):
    p = partial.reshape(NC, NROWS // 128, 128)

    def body(p_ref, o_ref):
        o_ref[...] = (p_ref[0] + p_ref[1]) * (1.0 / PIXEL_SCALE)

    out = pl.pallas_call(
        body,
        out_shape=jax.ShapeDtypeStruct((NROWS // 128, 128), jnp.float32),
    )(p)
    return out.reshape(2, 2, N_VERTS)


def kernel(x, vals, rows, cols):
    x_flat = x.reshape(-1)
    partial = _sc_partial(x_flat, vals, rows, cols)
    grad = _combine(partial)
    return jnp.moveaxis(grad, 0, -1)
